# baseline (speedup 1.0000x reference)
import jax
import jax.numpy as jnp
from jax import lax
from jax.experimental import pallas as pl
from jax.experimental.pallas import tpu as pltpu
from jax.experimental.pallas import tpu_sc as plsc

N_ROWS = 1000000
DIM = 32
BATCH = 16384

NC = 2
NS = 16
NW = NC * NS
RB = 31248
TAIL = N_ROWS - NW * RB
RMAX = RB + TAIL
CH = 512
COPY_PIECES = 6
PIECE = RB // COPY_PIECES
NVREG = BATCH // 16
FIN_CAP = BATCH + CH + 16

_SENTINEL = 0x7FFFFFF0


def _sc_set_kernel(mem_hbm, idx_hbm, val_hbm, out_hbm,
                   idx_v, tab_v, fin_idx_v, fin_pos_v,
                   chunk_idx_v, chunk_pos_v, rows_v,
                   copy_sem, g_sem, s_sem):
    w = lax.axis_index("s") * NC + lax.axis_index("c")
    lo = pl.multiple_of(w * RB, 8)
    nrows = jnp.where(w == NW - 1, RB + TAIL, RB)
    iota = lax.iota(jnp.int32, 16)

    copies = []
    for p in range(COPY_PIECES):
        base = pl.multiple_of(lo + p * PIECE, 8)
        copies.append(pltpu.async_copy(
            mem_hbm.at[pl.ds(base, PIECE)],
            out_hbm.at[pl.ds(base, PIECE)],
            copy_sem))

    @pl.when(w == NW - 1)
    def _tail_copy():
        pltpu.sync_copy(mem_hbm.at[pl.ds(NW * RB, TAIL)],
                        out_hbm.at[pl.ds(NW * RB, TAIL)])

    pltpu.sync_copy(idx_hbm, idx_v)

    def pass_a(i, carry):
        base = i * 16
        vec = idx_v[pl.ds(base, 16)]
        loc = vec - lo
        valid = (loc >= 0) & (loc < nrows)
        key = jnp.where(valid, (loc << 4) | iota, _SENTINEL)
        pos = jnp.where(valid, base + iota, -1)
        sk, sv = plsc.sort_key_val(key, pos)
        nbr = jnp.minimum(iota + 1, 15)
        knext = sk.at[nbr].get(mode="promise_in_bounds")
        run_last = ((sk >> 4) != (knext >> 4)) | (iota == 15)
        m = run_last & (sv >= 0)
        plsc.store_scatter(tab_v, [sk >> 4], sv, mask=m)
        return carry

    lax.fori_loop(0, NVREG, pass_a, jnp.int32(0))

    def pass_b(i, cnt):
        base = i * 16
        vec = idx_v[pl.ds(base, 16)]
        loc = vec - lo
        valid = (loc >= 0) & (loc < nrows)
        pos = base + iota
        t = plsc.load_gather(tab_v, [jnp.where(valid, loc, 0)], mask=valid)
        keep = valid & (t == pos)
        plsc.store_compressed(fin_idx_v.at[pl.ds(cnt, 16)], vec, mask=keep)
        plsc.store_compressed(fin_pos_v.at[pl.ds(cnt, 16)], pos, mask=keep)
        return cnt + jnp.max(plsc.all_reduce_population_count(keep))

    cnt2 = lax.fori_loop(0, NVREG, pass_b, jnp.int32(0))

    for c in copies:
        c.wait()

    @pl.when(cnt2 > 0)
    def _scatter():
        zeros = jnp.zeros((16,), jnp.int32)
        bi = fin_idx_v[pl.ds(0, 16)].at[zeros].get(mode="promise_in_bounds")
        bp = fin_pos_v[pl.ds(0, 16)].at[zeros].get(mode="promise_in_bounds")
        n_chunks = (cnt2 + CH - 1) // CH
        pad = n_chunks * CH - cnt2

        def pad_body(k, carry):
            off = cnt2 + k * 16
            fin_idx_v[pl.ds(off, 16)] = bi
            fin_pos_v[pl.ds(off, 16)] = bp
            return carry

        lax.fori_loop(0, (pad + 15) // 16, pad_body, jnp.int32(0))

        def chunk_body(c, carry):
            off = c * CH
            def stage(k, carry):
                chunk_idx_v[pl.ds(k * 16, 16)] = fin_idx_v[pl.ds(off + k * 16, 16)]
                chunk_pos_v[pl.ds(k * 16, 16)] = fin_pos_v[pl.ds(off + k * 16, 16)]
                return carry

            lax.fori_loop(0, CH // 16, stage, jnp.int32(0))
            pltpu.async_copy(val_hbm.at[chunk_pos_v], rows_v, g_sem).wait()
            pltpu.async_copy(rows_v, out_hbm.at[chunk_idx_v], s_sem).wait()
            return carry

        lax.fori_loop(0, n_chunks, chunk_body, jnp.int32(0))


@jax.jit
def _sc_set(memory, node_idxs, values):
    return pl.kernel(
        _sc_set_kernel,
        out_type=jax.ShapeDtypeStruct((N_ROWS, DIM), jnp.float32),
        mesh=plsc.VectorSubcoreMesh(core_axis_name="c", subcore_axis_name="s"),
        compiler_params=pltpu.CompilerParams(
            needs_layout_passes=False, use_tc_tiling_on_sc=False),
        scratch_types=[
            pltpu.VMEM((BATCH,), jnp.int32),
            pltpu.VMEM((RMAX,), jnp.int32),
            pltpu.VMEM((FIN_CAP,), jnp.int32),
            pltpu.VMEM((FIN_CAP,), jnp.int32),
            pltpu.VMEM((CH,), jnp.int32),
            pltpu.VMEM((CH,), jnp.int32),
            pltpu.VMEM((CH, DIM), jnp.float32),
            pltpu.SemaphoreType.DMA,
            pltpu.SemaphoreType.DMA,
            pltpu.SemaphoreType.DMA,
        ],
    )(memory, node_idxs, values)


def kernel(memory, node_idxs, values):
    return _sc_set(memory, node_idxs, values)
